# baseline (speedup 1.0000x reference)
import functools

import jax
import jax.numpy as jnp
from jax import lax
from jax.experimental import pallas as pl
from jax.experimental.pallas import tpu as pltpu
from jax.experimental.pallas import tpu_sc as plsc

NC = 2
NS = 16
NW = NC * NS
L = 16


def _build(M, K, NF):
    rpw = M // NW
    CW = 128
    C = rpw // CW
    P = 32
    NPIECE = CW // P
    TP = C * NPIECE

    mesh = plsc.VectorSubcoreMesh(core_axis_name="c", subcore_axis_name="s")

    @functools.partial(
        pl.kernel,
        out_type=jax.ShapeDtypeStruct((NF, M), jnp.float32),
        mesh=mesh,
        scratch_types=[
            pltpu.VMEM((NF,), jnp.int32),
            pltpu.VMEM((P, K), jnp.float32),
            pltpu.VMEM((P, K), jnp.float32),
            pltpu.VMEM((NF, CW), jnp.float32),
            pltpu.VMEM((NF, CW), jnp.float32),
            pltpu.SemaphoreType.DMA,
            pltpu.SemaphoreType.DMA,
            pltpu.SemaphoreType.DMA,
            pltpu.SemaphoreType.DMA,
        ],
        compiler_params=pltpu.CompilerParams(
            use_tc_tiling_on_sc=True,
            needs_layout_passes=False,
            disable_bounds_checks=True,
        ),
    )
    def k(x_hbm, idx_hbm, out_hbm, idxv, xpa, xpb, outa, outb,
          isa, isb, osa, osb):
        xps, outs = [xpa, xpb], [outa, outb]
        isems, osems = [isa, isb], [osa, osb]
        wid = lax.axis_index("s") * NC + lax.axis_index("c")
        row0 = wid * rpw
        pltpu.sync_copy(idx_hbm, idxv)
        iotas = [
            lax.iota(jnp.int32, L) + b * L for b in range(P // L)
        ]

        def start_in(p):
            b = p & 1
            return pltpu.async_copy(
                x_hbm.at[pl.ds(row0 + p * P, P)], xps[b], isems[b]
            )

        def start_out(c):
            b = c & 1
            return pltpu.async_copy(
                outs[b], out_hbm.at[:, pl.ds(row0 + c * CW, CW)], osems[b]
            )

        def compute_piece(p, c):
            xp = xps[p & 1]
            outv = outs[c & 1]
            col0 = (p % NPIECE) * P

            def jbody(j, _):
                jsplat = jnp.full((L,), j, jnp.int32)
                csplat = plsc.load_gather(idxv, [jsplat])
                for b in range(P // L):
                    vals = plsc.load_gather(xp, [iotas[b], csplat])
                    outv[j, pl.ds(col0 + b * L, L)] = vals
                return 0

            lax.fori_loop(0, NF, jbody, 0, unroll=2)

        h_in = [None] * TP
        h_out = [None] * C
        h_in[0] = start_in(0)
        for p in range(TP):
            c = p // NPIECE
            if p + 1 < TP:
                h_in[p + 1] = start_in(p + 1)
            h_in[p].wait()
            if p % NPIECE == 0 and c >= 2:
                h_out[c - 2].wait()
            compute_piece(p, c)
            if p % NPIECE == NPIECE - 1:
                h_out[c] = start_out(c)
        h_out[C - 2].wait()
        h_out[C - 1].wait()

    return k


def kernel(x, feature_indices):
    M, K = x.shape
    NF = feature_indices.shape[0]
    k = _build(M, K, NF)
    outT = k(x, feature_indices.astype(jnp.int32))
    return outT.T


# speedup vs baseline: 1.3436x; 1.3436x over previous
import functools

import jax
import jax.numpy as jnp
from jax import lax
from jax.experimental import pallas as pl
from jax.experimental.pallas import tpu as pltpu
from jax.experimental.pallas import tpu_sc as plsc

NC = 2
NS = 16
NW = NC * NS
L = 16


def _build(M, K, NF, NP):
    rpw = M // NW
    CW = 128
    C = rpw // CW
    P = 32
    NPIECE = CW // P
    TP = C * NPIECE
    NG = NP // L
    JT = (NF + 7) // 8
    KT = K // 128
    PW = P * K
    ITILES = M // 128

    mesh = plsc.VectorSubcoreMesh(core_axis_name="c", subcore_axis_name="s")

    @functools.partial(
        pl.kernel,
        out_type=jax.ShapeDtypeStruct((JT, ITILES * 1024), jnp.float32),
        mesh=mesh,
        scratch_types=[
            pltpu.VMEM((NP,), jnp.int32),
            pltpu.VMEM((NP,), jnp.int32),
            pltpu.VMEM((NP,), jnp.int32),
            pltpu.VMEM((NP,), jnp.int32),
            pltpu.VMEM((PW,), jnp.float32),
            pltpu.VMEM((PW,), jnp.float32),
            pltpu.VMEM((JT, 1024), jnp.float32),
            pltpu.VMEM((JT, 1024), jnp.float32),
            pltpu.SemaphoreType.DMA,
            pltpu.SemaphoreType.DMA,
            pltpu.SemaphoreType.DMA,
            pltpu.SemaphoreType.DMA,
        ],
        compiler_params=pltpu.CompilerParams(
            use_tc_tiling_on_sc=False,
            needs_layout_passes=False,
            disable_bounds_checks=True,
        ),
    )
    def k(x_hbm, idx_hbm, out_hbm, idxv, colt, rvt, cbt, xpa, xpb,
          outa, outb, isa, isb, osa, osb):
        xps, outs = [xpa, xpb], [outa, outb]
        isems, osems = [isa, isb], [osa, osb]
        wid = lax.axis_index("s") * NC + lax.axis_index("c")
        row0 = wid * rpw
        tc0 = row0 // CW
        iota = lax.iota(jnp.int32, L)

        pltpu.sync_copy(idx_hbm, idxv)
        for g in range(NG):
            v = idxv[pl.ds(g * L, L)]
            colt[pl.ds(g * L, L)] = (v >> 7) * 1024 + (v & 127)
            j0 = g * L if g < NG - 1 else NF - L
            jv = iota + j0
            rvt[pl.ds(g * L, L)] = jv >> 3
            cbt[pl.ds(g * L, L)] = (jv & 7) * 128

        def in_off(n):
            return (row0 + n * P) * K

        def issue_in(n, b):
            return pltpu.async_copy(
                x_hbm.at[pl.ds(in_off(n), PW)], xps[b], isems[b]
            )

        def wait_in(b):
            pltpu.make_async_copy(
                x_hbm.at[pl.ds(0, PW)], xps[b], isems[b]
            ).wait()

        def issue_out(ch, b):
            return pltpu.async_copy(
                outs[b],
                out_hbm.at[:, pl.ds((tc0 + ch) * 1024, 1024)],
                osems[b],
            )

        def wait_out(b):
            pltpu.make_async_copy(
                outs[b], out_hbm.at[:, pl.ds(0, 1024)], osems[b]
            ).wait()

        def compute_piece(xp, outv, col0):
            def gbody(g, _):
                g16 = pl.multiple_of(g * L, L)
                colp = colt[pl.ds(g16, L)]
                rv = rvt[pl.ds(g16, L)]
                cb = cbt[pl.ds(g16, L)] + col0

                def trbody(tr, carry):
                    gidx, cvec = carry
                    for s in range(8):
                        vals = plsc.load_gather(xp, [gidx + s * 128])
                        plsc.store_scatter(outv, [rv, cvec + s], vals)
                    return gidx + 4096, cvec + 8

                lax.fori_loop(0, P // 8, trbody, (colp, cb))
                return 0

            lax.fori_loop(0, NG, gbody, 0)

        issue_in(0, 0)

        def citer(it, _):
            for cc in range(2):
                ch = 2 * it + cc
                for q in range(NPIECE):
                    n = ch * NPIECE + q
                    wait_in(q & 1)

                    @pl.when(n + 1 < TP)
                    def _():
                        issue_in(n + 1, (q + 1) & 1)

                    if q == 0:
                        @pl.when(ch >= 2)
                        def _():
                            wait_out(cc)

                    compute_piece(xps[q & 1], outs[cc], q * P)
                issue_out(ch, cc)
            return 0

        lax.fori_loop(0, C // 2, citer, 0)
        wait_out(0)
        wait_out(1)

    return k


def kernel(x, feature_indices):
    M, K = x.shape
    NF = feature_indices.shape[0]
    G = NF // L
    rem = NF % L
    if rem:
        idx_pad = jnp.concatenate(
            [feature_indices[: G * L], feature_indices[NF - L :]]
        )
    else:
        idx_pad = feature_indices
    NP = idx_pad.shape[0]

    x1 = jnp.transpose(
        x.reshape(M // 8, 8, K // 128, 128), (0, 2, 1, 3)
    ).reshape(-1)

    k = _build(M, K, NF, NP)
    out1 = k(x1, idx_pad.astype(jnp.int32))

    JT = (NF + 7) // 8
    outT = jnp.transpose(
        out1.reshape(JT, M // 128, 8, 128), (0, 2, 1, 3)
    ).reshape(JT * 8, M)
    return outT[:NF].T


# speedup vs baseline: 2.2524x; 1.6764x over previous
import functools

import jax
import jax.numpy as jnp
from jax import lax
from jax.experimental import pallas as pl
from jax.experimental.pallas import tpu as pltpu
from jax.experimental.pallas import tpu_sc as plsc

NC = 2
NS = 16
NW = NC * NS
L = 16


def _build(M, K, NF, NP):
    rpw = M // NW
    R = 64
    C = rpw // R
    NG = NP // L
    PW = R * K

    mesh = plsc.VectorSubcoreMesh(core_axis_name="c", subcore_axis_name="s")

    @functools.partial(
        pl.kernel,
        out_type=jax.ShapeDtypeStruct((M, NF), jnp.float32),
        mesh=mesh,
        scratch_types=[
            pltpu.VMEM((NP,), jnp.int32),
            pltpu.VMEM((NP,), jnp.int32),
            pltpu.VMEM((PW,), jnp.float32),
            pltpu.VMEM((PW,), jnp.float32),
            pltpu.VMEM((R, NF), jnp.float32),
            pltpu.VMEM((R, NF), jnp.float32),
            pltpu.SemaphoreType.DMA,
            pltpu.SemaphoreType.DMA,
            pltpu.SemaphoreType.DMA,
            pltpu.SemaphoreType.DMA,
        ],
        compiler_params=pltpu.CompilerParams(
            use_tc_tiling_on_sc=True,
            needs_layout_passes=False,
            disable_bounds_checks=True,
        ),
    )
    def k(x_hbm, idx_hbm, out_hbm, idxv, colt, xpa, xpb, outa, outb,
          isa, isb, osa, osb):
        xps, outs = [xpa, xpb], [outa, outb]
        isems, osems = [isa, isb], [osa, osb]
        wid = lax.axis_index("s") * NC + lax.axis_index("c")
        row0 = wid * rpw

        pltpu.sync_copy(idx_hbm, idxv)
        for g in range(NG):
            v = idxv[pl.ds(g * L, L)]
            colt[pl.ds(g * L, L)] = (v >> 7) * 1024 + (v & 127)

        def issue_in(n, b):
            return pltpu.async_copy(
                x_hbm.at[pl.ds((row0 + n * R) * K, PW)], xps[b], isems[b]
            )

        def wait_in(b):
            pltpu.make_async_copy(
                x_hbm.at[pl.ds(0, PW)], xps[b], isems[b]
            ).wait()

        def issue_out(ch, b):
            return pltpu.async_copy(
                outs[b], out_hbm.at[pl.ds(row0 + ch * R, R)], osems[b]
            )

        def wait_out(b):
            pltpu.make_async_copy(
                outs[b], out_hbm.at[pl.ds(0, R)], osems[b]
            ).wait()

        def compute_chunk(xp, outv):
            def gbody(g, _):
                g16 = pl.multiple_of(g * L, L)
                colp = colt[pl.ds(g16, L)]
                off = jnp.minimum(g * L, NF - L)

                def trbody(tr, gidx):
                    r0 = tr * 8
                    for s in range(8):
                        vals = plsc.load_gather(xp, [gidx + s * 128])
                        outv[r0 + s, pl.ds(off, L)] = vals
                    return gidx + 4096

                lax.fori_loop(0, R // 8, trbody, colp)
                return 0

            lax.fori_loop(0, NG, gbody, 0)

        issue_in(0, 0)

        def citer(it, _):
            for cc in range(2):
                ch = 2 * it + cc
                wait_in(cc)

                @pl.when(ch + 1 < C)
                def _():
                    issue_in(ch + 1, cc ^ 1)

                @pl.when(ch >= 2)
                def _():
                    wait_out(cc)

                compute_chunk(xps[cc], outs[cc])
                issue_out(ch, cc)
            return 0

        lax.fori_loop(0, C // 2, citer, 0)
        wait_out(0)
        wait_out(1)

    return k


def kernel(x, feature_indices):
    M, K = x.shape
    NF = feature_indices.shape[0]
    G = NF // L
    rem = NF % L
    if rem:
        idx_pad = jnp.concatenate(
            [feature_indices[: G * L], feature_indices[NF - L :]]
        )
    else:
        idx_pad = feature_indices
    NP = idx_pad.shape[0]

    x1 = jnp.transpose(
        x.reshape(M // 8, 8, K // 128, 128), (0, 2, 1, 3)
    ).reshape(-1)

    k = _build(M, K, NF, NP)
    return k(x1, idx_pad.astype(jnp.int32))


# speedup vs baseline: 2.2664x; 1.0062x over previous
import functools

import jax
import jax.numpy as jnp
from jax import lax
from jax.experimental import pallas as pl
from jax.experimental.pallas import tpu as pltpu
from jax.experimental.pallas import tpu_sc as plsc

NC = 2
NS = 16
NW = NC * NS
L = 16


def _build(M, K, NF, NP):
    rpw = M // NW
    R = 64
    C = rpw // R
    NG = NP // L
    PW = R * K

    mesh = plsc.VectorSubcoreMesh(core_axis_name="c", subcore_axis_name="s")

    @functools.partial(
        pl.kernel,
        out_type=jax.ShapeDtypeStruct((M, NF), jnp.float32),
        mesh=mesh,
        scratch_types=[
            pltpu.VMEM((NP,), jnp.int32),
            pltpu.VMEM((NP,), jnp.int32),
            pltpu.VMEM((PW,), jnp.float32),
            pltpu.VMEM((PW,), jnp.float32),
            pltpu.VMEM((R, NF), jnp.float32),
            pltpu.VMEM((R, NF), jnp.float32),
            pltpu.SemaphoreType.DMA,
            pltpu.SemaphoreType.DMA,
            pltpu.SemaphoreType.DMA,
            pltpu.SemaphoreType.DMA,
        ],
        compiler_params=pltpu.CompilerParams(
            use_tc_tiling_on_sc=True,
            needs_layout_passes=False,
            disable_bounds_checks=True,
        ),
    )
    def k(x_hbm, idx_hbm, out_hbm, idxv, colt, xpa, xpb, outa, outb,
          isa, isb, osa, osb):
        xps, outs = [xpa, xpb], [outa, outb]
        isems, osems = [isa, isb], [osa, osb]
        wid = lax.axis_index("s") * NC + lax.axis_index("c")
        row0 = wid * rpw

        def prefetch_first(n, b):
            return pltpu.async_copy(
                x_hbm.at[pl.ds((row0 + n * R) * K, PW)], xps[b], isems[b]
            )

        prefetch_first(0, 0)

        pltpu.sync_copy(idx_hbm, idxv)
        for g in range(NG):
            v = idxv[pl.ds(g * L, L)]
            colt[pl.ds(g * L, L)] = (v >> 7) * 1024 + (v & 127)

        def issue_in(n, b):
            return pltpu.async_copy(
                x_hbm.at[pl.ds((row0 + n * R) * K, PW)], xps[b], isems[b]
            )

        def wait_in(b):
            pltpu.make_async_copy(
                x_hbm.at[pl.ds(0, PW)], xps[b], isems[b]
            ).wait()

        def issue_out(ch, b):
            return pltpu.async_copy(
                outs[b], out_hbm.at[pl.ds(row0 + ch * R, R)], osems[b]
            )

        def wait_out(b):
            pltpu.make_async_copy(
                outs[b], out_hbm.at[pl.ds(0, R)], osems[b]
            ).wait()

        def compute_chunk(xp, outv):
            def gbody(g, _):
                g16 = pl.multiple_of(g * L, L)
                colp = colt[pl.ds(g16, L)]
                off = jnp.minimum(g * L, NF - L)

                def trbody(tr, gidx):
                    r0 = tr * 8
                    for s in range(8):
                        vals = plsc.load_gather(xp, [gidx + s * 128])
                        outv[r0 + s, pl.ds(off, L)] = vals
                    return gidx + 4096

                lax.fori_loop(0, R // 8, trbody, colp, unroll=2)
                return 0

            lax.fori_loop(0, NG, gbody, 0)

        def citer(it, _):
            for cc in range(2):
                ch = 2 * it + cc
                wait_in(cc)

                @pl.when(ch + 1 < C)
                def _():
                    issue_in(ch + 1, cc ^ 1)

                @pl.when(ch >= 2)
                def _():
                    wait_out(cc)

                compute_chunk(xps[cc], outs[cc])
                issue_out(ch, cc)
            return 0

        lax.fori_loop(0, C // 2, citer, 0)
        wait_out(0)
        wait_out(1)

    return k


def kernel(x, feature_indices):
    M, K = x.shape
    NF = feature_indices.shape[0]
    G = NF // L
    rem = NF % L
    if rem:
        idx_pad = jnp.concatenate(
            [feature_indices[: G * L], feature_indices[NF - L :]]
        )
    else:
        idx_pad = feature_indices
    NP = idx_pad.shape[0]

    x1 = jnp.transpose(
        x.reshape(M // 8, 8, K // 128, 128), (0, 2, 1, 3)
    ).reshape(-1)

    k = _build(M, K, NF, NP)
    return k(x1, idx_pad.astype(jnp.int32))


# speedup vs baseline: 3.2213x; 1.4213x over previous
import functools

import jax
import jax.numpy as jnp
from jax import lax
from jax.experimental import pallas as pl
from jax.experimental.pallas import tpu as pltpu
from jax.experimental.pallas import tpu_sc as plsc

NC = 2
NS = 16
NW = NC * NS
L = 16


def _build(M, K, NF, NP):
    rpw = M // NW
    R = 64
    C = rpw // R
    NG = NP // L
    PW = R * K

    mesh = plsc.VectorSubcoreMesh(core_axis_name="c", subcore_axis_name="s")

    @functools.partial(
        pl.kernel,
        out_type=jax.ShapeDtypeStruct((M, NF), jnp.float32),
        mesh=mesh,
        scratch_types=[
            pltpu.VMEM((NP,), jnp.int32),
            pltpu.VMEM((NP,), jnp.int32),
            pltpu.VMEM((PW,), jnp.float32),
            pltpu.VMEM((PW,), jnp.float32),
            pltpu.VMEM((R, NF), jnp.float32),
            pltpu.VMEM((R, NF), jnp.float32),
            pltpu.SemaphoreType.DMA,
            pltpu.SemaphoreType.DMA,
            pltpu.SemaphoreType.DMA,
            pltpu.SemaphoreType.DMA,
        ],
        compiler_params=pltpu.CompilerParams(
            use_tc_tiling_on_sc=True,
            needs_layout_passes=False,
            disable_bounds_checks=True,
        ),
    )
    def k(x_hbm, idx_hbm, out_hbm, idxv, colt, xpa, xpb, outa, outb,
          isa, isb, osa, osb):
        xps, outs = [xpa, xpb], [outa, outb]
        isems, osems = [isa, isb], [osa, osb]
        wid = lax.axis_index("s") * NC + lax.axis_index("c")
        row0 = wid * rpw

        def prefetch_first(n, b):
            return pltpu.async_copy(
                x_hbm.at[pl.ds((row0 + n * R) * K, PW)], xps[b], isems[b]
            )

        prefetch_first(0, 0)

        pltpu.sync_copy(idx_hbm, idxv)
        for g in range(NG):
            v = idxv[pl.ds(g * L, L)]
            colt[pl.ds(g * L, L)] = (v >> 7) * 1024 + (v & 127)

        def issue_in(n, b):
            return pltpu.async_copy(
                x_hbm.at[pl.ds((row0 + n * R) * K, PW)], xps[b], isems[b]
            )

        def wait_in(b):
            pltpu.make_async_copy(
                x_hbm.at[pl.ds(0, PW)], xps[b], isems[b]
            ).wait()

        def issue_out(ch, b):
            return pltpu.async_copy(
                outs[b], out_hbm.at[pl.ds(row0 + ch * R, R)], osems[b]
            )

        def wait_out(b):
            pltpu.make_async_copy(
                outs[b], out_hbm.at[pl.ds(0, R)], osems[b]
            ).wait()

        def compute_chunk(xp, outv):
            @plsc.parallel_loop(0, NG)
            def gbody(g):
                g16 = pl.multiple_of(g * L, L)
                colp = colt[pl.ds(g16, L)]
                off = jnp.minimum(g * L, NF - L)

                @plsc.parallel_loop(0, R // 8, carry=colp, unroll=2)
                def trbody(tr, gidx):
                    r0 = tr * 8
                    for s in range(8):
                        vals = plsc.load_gather(xp, [gidx + s * 128])
                        outv[r0 + s, pl.ds(off, L)] = vals
                    return gidx + 4096

        def citer(it, _):
            for cc in range(2):
                ch = 2 * it + cc
                wait_in(cc)

                @pl.when(ch + 1 < C)
                def _():
                    issue_in(ch + 1, cc ^ 1)

                @pl.when(ch >= 2)
                def _():
                    wait_out(cc)

                compute_chunk(xps[cc], outs[cc])
                issue_out(ch, cc)
            return 0

        lax.fori_loop(0, C // 2, citer, 0)
        wait_out(0)
        wait_out(1)

    return k


def kernel(x, feature_indices):
    M, K = x.shape
    NF = feature_indices.shape[0]
    G = NF // L
    rem = NF % L
    if rem:
        idx_pad = jnp.concatenate(
            [feature_indices[: G * L], feature_indices[NF - L :]]
        )
    else:
        idx_pad = feature_indices
    NP = idx_pad.shape[0]

    x1 = jnp.transpose(
        x.reshape(M // 8, 8, K // 128, 128), (0, 2, 1, 3)
    ).reshape(-1)

    k = _build(M, K, NF, NP)
    return k(x1, idx_pad.astype(jnp.int32))
